# baseline (speedup 1.0000x reference)
import jax
import jax.numpy as jnp
from jax.experimental import pallas as pl


def _gcn_body(x_ref, adj_ref, w1_ref, b1_ref, w2_ref, b2_ref, out_ref):
    s1 = jnp.dot(x_ref[...], w1_ref[...], preferred_element_type=jnp.float32)
    h1 = jax.nn.relu(
        jnp.dot(adj_ref[...], s1, preferred_element_type=jnp.float32) + b1_ref[...]
    )
    s2 = jnp.dot(h1, w2_ref[...], preferred_element_type=jnp.float32)
    h2 = jax.nn.relu(
        jnp.dot(adj_ref[...], s2, preferred_element_type=jnp.float32) + b2_ref[...]
    )
    out_ref[...] = h2


def _head_body(flat_ref, fc1w_ref, fc1b_ref, fc2w_ref, fc2b_ref, fc3w_ref,
               fc3b_ref, out_ref):
    dn = (((1,), (1,)), ((), ()))
    a1 = jax.nn.relu(
        jax.lax.dot_general(flat_ref[...], fc1w_ref[...], dn,
                            preferred_element_type=jnp.float32) + fc1b_ref[...]
    )
    a2 = jax.nn.relu(
        jax.lax.dot_general(a1, fc2w_ref[...], dn,
                            preferred_element_type=jnp.float32) + fc2b_ref[...]
    )
    a3 = jnp.sum(a2 * fc3w_ref[...], axis=1, keepdims=True) + fc3b_ref[...]
    out_ref[...] = jax.nn.sigmoid(a3)


def kernel(x, adj, W1, b1, W2, b2, fc1_w, fc1_b, fc2_w, fc2_b, fc3_w, fc3_b):
    n, nclass = adj.shape[0], W2.shape[1]

    h2 = pl.pallas_call(
        _gcn_body,
        out_shape=jax.ShapeDtypeStruct((n, nclass), jnp.float32),
    )(x, adj, W1, b1.reshape(1, -1), W2, b2.reshape(1, -1))

    flat = h2.reshape(1, -1)
    y = pl.pallas_call(
        _head_body,
        out_shape=jax.ShapeDtypeStruct((1, 1), jnp.float32),
    )(flat, fc1_w, fc1_b.reshape(1, -1), fc2_w, fc2_b.reshape(1, -1),
      fc3_w, fc3_b.reshape(1, -1))

    return y.reshape(1)
